# Initial kernel scaffold; baseline (speedup 1.0000x reference)
import jax
import jax.numpy as jnp
from jax.experimental import pallas as pl


def kernel(x, edge_index, W1, b1, W2, b2, W3, b3, Wc, bc):
    raise NotImplementedError("write your pallas kernel here")


# speedup vs baseline: 19.1186x; 19.1186x over previous
import functools

import jax
import jax.numpy as jnp
from jax import lax
from jax.experimental import pallas as pl
from jax.experimental.pallas import tpu as pltpu
from jax.experimental.pallas import tpu_sc as plsc

N = 10000
D = 128
E = 320000
NUM_CLASSES = 16

NC = 2
NS = 16
CHUNK = 128
NPAD = 10240
ROWS_PER_SUB = NPAD // NS
EPW = ((E + NC * NS * CHUNK - 1) // (NC * NS * CHUNK)) * CHUNK
EPAD = NC * NS * EPW
TRASH = N


def _sc_mesh():
    return plsc.VectorSubcoreMesh(core_axis_name="c", subcore_axis_name="s")


_SC_PARAMS = pltpu.CompilerParams(use_tc_tiling_on_sc=False)


def _make_edge_pass(w):
    @functools.partial(
        pl.kernel,
        mesh=_sc_mesh(),
        compiler_params=_SC_PARAMS,
        out_type=jax.ShapeDtypeStruct((NC, NPAD, w), jnp.float32),
        scratch_types=[
            pltpu.VMEM((CHUNK,), jnp.int32),
            pltpu.VMEM((CHUNK,), jnp.int32),
            pltpu.VMEM((CHUNK, w), jnp.float32),
            pltpu.VMEM_SHARED((NPAD, w), jnp.float32),
            pltpu.SemaphoreType.DMA,
        ],
    )
    def edge_pass(src_hbm, dst_hbm, y_hbm, z_hbm, out_hbm,
                  src_v, dst_v, rows_v, acc_sh, sem):
        c = lax.axis_index("c")
        s = lax.axis_index("s")
        wid = c * NS + s
        r0 = s * ROWS_PER_SUB
        pltpu.sync_copy(z_hbm.at[pl.ds(r0, ROWS_PER_SUB)],
                        acc_sh.at[pl.ds(r0, ROWS_PER_SUB)])
        plsc.subcore_barrier()
        base0 = wid * EPW

        @pl.loop(0, EPW, step=CHUNK)
        def _(i):
            b = base0 + i
            pltpu.sync_copy(src_hbm.at[pl.ds(b, CHUNK)], src_v)
            pltpu.sync_copy(dst_hbm.at[pl.ds(b, CHUNK)], dst_v)
            pltpu.async_copy(y_hbm.at[src_v], rows_v, sem).wait()
            pltpu.sync_copy(rows_v, acc_sh.at[dst_v], add=True)

        plsc.subcore_barrier()
        pltpu.sync_copy(acc_sh.at[pl.ds(r0, ROWS_PER_SUB)],
                        out_hbm.at[c, pl.ds(r0, ROWS_PER_SUB)])

    return edge_pass


def _make_deg_pass():
    @functools.partial(
        pl.kernel,
        mesh=_sc_mesh(),
        compiler_params=_SC_PARAMS,
        out_type=jax.ShapeDtypeStruct((NC, NPAD, 1), jnp.float32),
        scratch_types=[
            pltpu.VMEM((CHUNK,), jnp.int32),
            pltpu.VMEM((CHUNK, 1), jnp.float32),
            pltpu.VMEM_SHARED((NPAD, 1), jnp.float32),
        ],
    )
    def deg_pass(dst_hbm, ones_hbm, z_hbm, out_hbm, dst_v, ones_v, acc_sh):
        c = lax.axis_index("c")
        s = lax.axis_index("s")
        wid = c * NS + s
        r0 = s * ROWS_PER_SUB
        pltpu.sync_copy(ones_hbm, ones_v)
        pltpu.sync_copy(z_hbm.at[pl.ds(r0, ROWS_PER_SUB)],
                        acc_sh.at[pl.ds(r0, ROWS_PER_SUB)])
        plsc.subcore_barrier()
        base0 = wid * EPW

        @pl.loop(0, EPW, step=CHUNK)
        def _(i):
            b = base0 + i
            pltpu.sync_copy(dst_hbm.at[pl.ds(b, CHUNK)], dst_v)
            pltpu.sync_copy(ones_v, acc_sh.at[dst_v], add=True)

        plsc.subcore_barrier()
        pltpu.sync_copy(acc_sh.at[pl.ds(r0, ROWS_PER_SUB)],
                        out_hbm.at[c, pl.ds(r0, ROWS_PER_SUB)])

    return deg_pass


def _tc_matmul(x, W):
    def body(x_ref, w_ref, o_ref):
        o_ref[...] = jnp.dot(x_ref[...], w_ref[...],
                             preferred_element_type=jnp.float32)

    return pl.pallas_call(
        body,
        out_shape=jax.ShapeDtypeStruct((x.shape[0], W.shape[1]), jnp.float32),
    )(x, W)


def _tc_prep1(deg_parts, xw):
    def body(deg_ref, xw_ref, y_ref, dinv_ref):
        dp = deg_ref[...]
        dinv = lax.rsqrt(dp[0] + dp[1] + 1.0)[:N]
        dinv_ref[...] = dinv
        y_ref[...] = xw_ref[...] * dinv

    return pl.pallas_call(
        body,
        out_shape=(
            jax.ShapeDtypeStruct((N, xw.shape[1]), jnp.float32),
            jax.ShapeDtypeStruct((N, 1), jnp.float32),
        ),
    )(deg_parts, xw)


def _tc_mid(acc_parts, y, dinv, Wn, b):
    def body(acc_ref, y_ref, dinv_ref, w_ref, b_ref, o_ref):
        a = acc_ref[...]
        tot = a[0, :N] + a[1, :N] + y_ref[...]
        di = dinv_ref[...]
        h = jnp.tanh(tot * di + b_ref[...])
        o_ref[...] = jnp.dot(h, w_ref[...],
                             preferred_element_type=jnp.float32) * di

    return pl.pallas_call(
        body,
        out_shape=jax.ShapeDtypeStruct((N, Wn.shape[1]), jnp.float32),
    )(acc_parts, y, dinv, Wn, b)


def _tc_final(acc_parts, y, dinv, b3, Wc, bc):
    def body(acc_ref, y_ref, dinv_ref, b3_ref, wc_ref, bc_ref, o_ref, h_ref):
        a = acc_ref[...]
        tot = a[0, :N] + a[1, :N] + y_ref[...]
        h = jnp.tanh(tot * dinv_ref[...] + b3_ref[...])
        h_ref[...] = h
        o_ref[...] = jnp.dot(h, wc_ref[...],
                             preferred_element_type=jnp.float32) + bc_ref[...]

    return pl.pallas_call(
        body,
        out_shape=(
            jax.ShapeDtypeStruct((N, NUM_CLASSES), jnp.float32),
            jax.ShapeDtypeStruct((N, 2), jnp.float32),
        ),
    )(acc_parts, y, dinv, b3, Wc, bc)


def kernel(x, edge_index, W1, b1, W2, b2, W3, b3, Wc, bc):
    src = edge_index[0]
    dst = edge_index[1]
    pad = EPAD - E
    src_p = jnp.concatenate([src, jnp.zeros((pad,), jnp.int32)])
    dst_p = jnp.concatenate([dst, jnp.full((pad,), TRASH, jnp.int32)])

    z4 = jnp.zeros((NPAD, 4), jnp.float32)
    z2 = jnp.zeros((NPAD, 2), jnp.float32)
    z1 = jnp.zeros((NPAD, 1), jnp.float32)
    ones = jnp.ones((CHUNK, 1), jnp.float32)

    deg_pass = _make_deg_pass()
    pass4 = _make_edge_pass(4)
    pass2 = _make_edge_pass(2)

    deg_parts = deg_pass(dst_p, ones, z1)
    xw1 = _tc_matmul(x, W1)
    y1, dinv = _tc_prep1(deg_parts, xw1)

    acc1 = pass4(src_p, dst_p, y1, z4)
    y2 = _tc_mid(acc1, y1, dinv, W2, b1.reshape(1, 4))

    acc2 = pass4(src_p, dst_p, y2, z4)
    y3 = _tc_mid(acc2, y2, dinv, W3, b2.reshape(1, 4))

    acc3 = pass2(src_p, dst_p, y3, z2)
    out, h = _tc_final(acc3, y3, dinv, b3.reshape(1, 2), Wc, bc.reshape(1, NUM_CLASSES))
    return (out, h)
